# Initial kernel scaffold; baseline (speedup 1.0000x reference)
import jax
import jax.numpy as jnp
from jax.experimental import pallas as pl


def kernel(hypo_scores, logits, beam_width):
    raise NotImplementedError("write your pallas kernel here")


# speedup vs baseline: 18.3185x; 18.3185x over previous
import functools

import jax
import jax.numpy as jnp
import numpy as np
from jax import lax
from jax.experimental import pallas as pl
from jax.experimental.pallas import tpu as pltpu
from jax.experimental.pallas import tpu_sc as plsc

BEAMS = 64
VOCAB = 100000
K = 16
L = 16
NC, NS = 2, 16
NW = NC * NS
ROWS_PER_W = BEAMS // NW
NVREG = VOCAB // L
NEG = np.float32(-3.0e38)


def _sc_body(logits_hbm, stats_hbm, topv_hbm, topi_hbm,
             rowbuf, stage_f, stage_i, stage_s):
  cid = lax.axis_index("c")
  sid = lax.axis_index("s")
  wid = sid * NC + cid
  lane = lax.iota(jnp.int32, L)

  for j in range(ROWS_PER_W):
    r = wid * ROWS_PER_W + j
    pltpu.sync_copy(logits_hbm.at[r], rowbuf)

    def p1(i, acc):
      return jnp.maximum(acc, rowbuf[pl.ds(i * L, L)])
    m = jnp.max(lax.fori_loop(0, NVREG, p1, jnp.full((L,), NEG, jnp.float32)))

    def p2(i, acc):
      return acc + jnp.exp(rowbuf[pl.ds(i * L, L)] - m)
    s = jnp.sum(lax.fori_loop(0, NVREG, p2, jnp.zeros((L,), jnp.float32)))

    vlast = rowbuf[pl.ds(VOCAB - L, L)]
    xb = jnp.max(jnp.where(lane == L - 1, vlast, NEG))

    def p3(i, carry):
      tv, ti, th = carry
      v = rowbuf[pl.ds(i * L, L)]
      kill = jnp.logical_and(lane == L - 1, i == NVREG - 1)
      v = jnp.where(kill, NEG, v)
      hit = jnp.max(v) > th

      def merge(c):
        tv0, ti0, _ = c
        idx = i * L + lane
        sv, si = plsc.sort_key_val(v, idx, descending=True)
        rv = lax.rev(sv, (0,))
        ri = lax.rev(si, (0,))
        take_cur = tv0 >= rv
        mv = jnp.where(take_cur, tv0, rv)
        mi = jnp.where(take_cur, ti0, ri)
        ntv, nti = plsc.sort_key_val(mv, mi, descending=True)
        return ntv, nti, jnp.min(ntv)

      return lax.cond(hit, merge, lambda c: c, (tv, ti, th))

    tv, ti, _ = lax.fori_loop(
        0, NVREG, p3,
        (jnp.full((L,), NEG, jnp.float32), jnp.zeros((L,), jnp.int32), NEG))

    statvec = jnp.where(lane == 0, m,
                        jnp.where(lane == 1, s,
                                  jnp.where(lane == 2, xb, 0.0)))
    stage_s[...] = statvec
    pltpu.sync_copy(stage_s, stats_hbm.at[r])
    stage_f[...] = tv
    pltpu.sync_copy(stage_f, topv_hbm.at[r])
    stage_i[...] = ti
    pltpu.sync_copy(stage_i, topi_hbm.at[r])


@jax.jit
def _sc_call(logits):
  mesh = plsc.VectorSubcoreMesh(core_axis_name="c", subcore_axis_name="s")
  f = pl.kernel(
      _sc_body,
      out_type=(
          jax.ShapeDtypeStruct((BEAMS, L), jnp.float32),
          jax.ShapeDtypeStruct((BEAMS, K), jnp.float32),
          jax.ShapeDtypeStruct((BEAMS, K), jnp.int32),
      ),
      mesh=mesh,
      compiler_params=pltpu.CompilerParams(needs_layout_passes=False),
      scratch_types=[
          pltpu.VMEM((VOCAB,), jnp.float32),
          pltpu.VMEM((L,), jnp.float32),
          pltpu.VMEM((L,), jnp.int32),
          pltpu.VMEM((L,), jnp.float32),
      ],
  )
  return f(logits)


def _merge_body(hypo_ref, stats_ref, topv_ref, topi_ref,
                scores_ref, rows_ref, toks_ref, blank_ref):
  stats = stats_ref[...]
  m = stats[:, 0:1]
  s = stats[:, 1:2]
  xb = stats[:, 2:3]
  hypo = hypo_ref[...]
  lse = m + jnp.log(s)
  off = hypo - lse
  adj = topv_ref[...] + off
  blank_ref[...] = hypo + xb - lse

  rowi = lax.broadcasted_iota(jnp.int32, (BEAMS, K), 0)
  coli = lax.broadcasted_iota(jnp.int32, (BEAMS, K), 1)
  flat = rowi * K + coli
  topi = topi_ref[...]
  big = jnp.int32(BEAMS * K)
  lane16 = lax.broadcasted_iota(jnp.int32, (1, K), 1)
  sc = jnp.zeros((1, K), jnp.float32)
  ro = jnp.zeros((1, K), jnp.int32)
  tk = jnp.zeros((1, K), jnp.int32)
  for r in range(K):
    mv = jnp.max(adj)
    p = jnp.min(jnp.where(adj == mv, flat, big))
    row = p // K
    tok = jnp.sum(jnp.where(flat == p, topi, 0))
    sc = jnp.where(lane16 == r, mv, sc)
    ro = jnp.where(lane16 == r, row, ro)
    tk = jnp.where(lane16 == r, tok, tk)
    adj = jnp.where(flat == p, NEG, adj)
  scores_ref[...] = sc
  rows_ref[...] = ro
  toks_ref[...] = tk


@jax.jit
def _merge_call(hypo2, stats, topv, topi):
  return pl.pallas_call(
      _merge_body,
      out_shape=(
          jax.ShapeDtypeStruct((1, K), jnp.float32),
          jax.ShapeDtypeStruct((1, K), jnp.int32),
          jax.ShapeDtypeStruct((1, K), jnp.int32),
          jax.ShapeDtypeStruct((BEAMS, 1), jnp.float32),
      ),
  )(hypo2, stats, topv, topi)


def kernel(hypo_scores, logits, beam_width):
  del beam_width
  stats, topv, topi = _sc_call(logits)
  sc, ro, tk, blank = _merge_call(hypo_scores.reshape(BEAMS, 1),
                                  stats, topv, topi)
  return (sc.reshape(K), ro.reshape(K), tk.reshape(K), blank.reshape(BEAMS))


# speedup vs baseline: 145.1583x; 7.9241x over previous
import jax
import jax.numpy as jnp
import numpy as np
from jax import lax
from jax.experimental import pallas as pl
from jax.experimental.pallas import tpu as pltpu
from jax.experimental.pallas import tpu_sc as plsc

BEAMS = 64
VOCAB = 100000
K = 16
L = 16
NC, NS = 2, 16
NW = NC * NS
ROWS_PER_W = BEAMS // NW
NVREG = VOCAB // L
U = 10
NBLK = NVREG // U
CAP = 2048
CHW = 19840
CHUNK_BLKS = [CHW // (U * 16)] * 5 + [(VOCAB - 5 * CHW) // (U * 16)]
NEG = np.float32(-3.0e38)


def _sc_body(logits_hbm, tail_hbm, stats_hbm, topv_hbm, topi_hbm,
             rowbuf, bmbuf, candv, candc, blkids, stage_f, stage_i, stage_s,
             dmasem):
  cid = lax.axis_index("c")
  sid = lax.axis_index("s")
  wid = sid * NC + cid
  lane = lax.iota(jnp.int32, L)
  splat15 = jnp.full((L,), L - 1, jnp.int32)

  for j in range(ROWS_PER_W):
    r = wid * ROWS_PER_W + j

    nchunk = len(CHUNK_BLKS)
    offs = [0]
    for nb in CHUNK_BLKS:
      offs.append(offs[-1] + nb * U * L)
    def start_chunk(c):
      if c == nchunk - 1:
        return pltpu.async_copy(
            tail_hbm.at[pl.ds(r * 1024, 1024)],
            rowbuf.at[pl.ds(VOCAB - 1024, 1024)], dmasem)
      return pltpu.async_copy(
          logits_hbm.at[r, pl.ds(offs[c], offs[c + 1] - offs[c])],
          rowbuf.at[pl.ds(offs[c], offs[c + 1] - offs[c])], dmasem)

    cps = [None] * nchunk
    for c in range(2):
      cps[c] = start_chunk(c)
    a1 = jnp.full((L,), NEG, jnp.float32)
    a2 = jnp.full((L,), NEG, jnp.float32)
    for c in range(nchunk):
      cps[c].wait()
      if c + 2 < nchunk:
        cps[c + 2] = start_chunk(c + 2)

      blk0 = offs[c] // (U * L)

      def pA(i, carry, blk0=blk0):
        t1, t2 = carry
        blk = blk0 + i
        base = blk * (U * L)
        vs = [rowbuf[pl.ds(base + u * L, L)] for u in range(U)]
        bm = vs[0]
        for u in range(1, U):
          bm = jnp.maximum(bm, vs[u])
        bmbuf[pl.ds(blk * L, L)] = bm
        hi = bm > t1
        t2 = jnp.where(hi, t1, jnp.maximum(t2, bm))
        t1 = jnp.where(hi, bm, t1)
        return t1, t2

      a1, a2 = lax.fori_loop(0, CHUNK_BLKS[c], pA, (a1, a2))

    m = jnp.max(a1)
    s1, _ = plsc.sort_key_val(a1, lane, descending=True)
    s2, _ = plsc.sort_key_val(a2, lane, descending=True)
    t_lo = jnp.min(jnp.maximum(s1, lax.rev(s2, (0,))))
    tvec = jnp.broadcast_to(t_lo, (L,))

    def pB(i, acc):
      a0, a1 = acc
      base = i * (U * L)
      for u in range(U):
        e = jnp.exp(rowbuf[pl.ds(base + u * L, L)] - m)
        if u % 2 == 0:
          a0 = a0 + e
        else:
          a1 = a1 + e
      return a0, a1
    a0, a1 = lax.fori_loop(0, NBLK, pB, (jnp.zeros((L,), jnp.float32),
                                         jnp.zeros((L,), jnp.float32)))
    s = jnp.sum(a0 + a1)

    vlast = rowbuf[pl.ds(VOCAB - L, L)]
    xb = jnp.max(jnp.where(lane == L - 1, vlast, NEG))
    rowbuf[pl.ds(VOCAB - L, L)] = jnp.where(lane == L - 1, NEG, vlast)

    def c0(i, bptr):
      bm = bmbuf[pl.ds(i * L, L)]
      mask = bm >= tvec
      pc = plsc.all_reduce_population_count(mask)
      ff = plsc.all_reduce_ffs(mask)
      first = jnp.logical_and(lane == ff, pc > 0)
      plsc.store_scatter(blkids, [bptr], jnp.broadcast_to(i, (L,)),
                         mask=first)
      return bptr + jnp.minimum(pc, 1)

    bptr = lax.fori_loop(0, NBLK, c0, jnp.zeros((L,), jnp.int32))
    blkcnt = bptr[0]

    def c1(i, cptr):
      chunk = blkids[pl.ds((i // L) * L, L)]
      bid_splat = chunk[jnp.broadcast_to(i % L, (L,))]
      bid = bid_splat[0]
      base = bid * (U * L)
      colbase = bid_splat * (U * L) + lane
      for u in range(U):
        v = rowbuf[pl.ds(base + u * L, L)]
        mask = v >= tvec
        cs = plsc.cumsum(mask.astype(jnp.int32))
        pc = plsc.all_reduce_population_count(mask)
        idx = jnp.minimum(cptr + cs - 1, CAP - 1)
        plsc.store_scatter(candv, [idx], v, mask=mask)
        plsc.store_scatter(candc, [idx], colbase + u * L, mask=mask)
        cptr = cptr + pc
      return cptr

    cptr = lax.fori_loop(0, blkcnt, c1, jnp.zeros((L,), jnp.int32))
    candcnt = cptr[0]
    tail = jnp.minimum(cptr + lane, CAP - 1)
    plsc.store_scatter(candv, [tail], jnp.full((L,), NEG, jnp.float32))
    bad = jnp.logical_or(candcnt < K, candcnt > CAP - L)

    def fmerge(c, carry):
      tv0, ti0 = carry
      v = candv[pl.ds(c * L, L)]
      cc = candc[pl.ds(c * L, L)]
      sv, si = plsc.sort_key_val(v, cc, descending=True)
      rv = lax.rev(sv, (0,))
      ri = lax.rev(si, (0,))
      take_cur = tv0 >= rv
      mv = jnp.where(take_cur, tv0, rv)
      mi = jnp.where(take_cur, ti0, ri)
      ntv, nti = plsc.sort_key_val(mv, mi, descending=True)
      return ntv, nti

    nch = jnp.where(bad, 0, (candcnt + L - 1) >> 4)
    tv, ti = lax.fori_loop(
        0, nch, fmerge,
        (jnp.full((L,), NEG, jnp.float32), jnp.zeros((L,), jnp.int32)))

    badv = jnp.broadcast_to(bad, (L,))

    def fb(i, carry):
      tv0, ti0, th0 = carry
      base = i * (U * L)
      vs = [rowbuf[pl.ds(base + u * L, L)] for u in range(U)]
      bm = vs[0]
      for u in range(1, U):
        bm = jnp.maximum(bm, vs[u])
      cnt = plsc.all_reduce_population_count(bm > th0)

      def scan_block(c):
        tv1, ti1, th1 = c
        for u in range(U):
          v = vs[u]
          cu = plsc.all_reduce_population_count(v > th1)

          def merge(c2, v=v, u=u):
            tv2, ti2, _ = c2
            idx = (i * U + u) * L + lane
            sv, si = plsc.sort_key_val(v, idx, descending=True)
            rv = lax.rev(sv, (0,))
            ri = lax.rev(si, (0,))
            take_cur = tv2 >= rv
            mv = jnp.where(take_cur, tv2, rv)
            mi = jnp.where(take_cur, ti2, ri)
            ntv, nti = plsc.sort_key_val(mv, mi, descending=True)
            return ntv, nti, jnp.min(ntv)

          tv1, ti1, th1 = lax.cond(cu[0] > 0, merge, lambda c2: c2,
                                   (tv1, ti1, th1))
        return tv1, ti1, th1

      return lax.cond(cnt[0] > 0, scan_block, lambda c: c, (tv0, ti0, th0))

    ntrip = jnp.where(bad, NBLK, 0)
    tv, ti, _ = lax.fori_loop(
        0, ntrip, fb,
        (jnp.where(badv, NEG, tv), jnp.where(badv, 0, ti),
         jnp.float32(NEG)))

    statvec = jnp.where(lane == 0, m,
                        jnp.where(lane == 1, s,
                                  jnp.where(lane == 2, xb, 0.0)))
    stage_s[...] = statvec
    pltpu.sync_copy(stage_s, stats_hbm.at[r])
    stage_f[...] = tv
    pltpu.sync_copy(stage_f, topv_hbm.at[r])
    stage_i[...] = ti
    pltpu.sync_copy(stage_i, topi_hbm.at[r])


@jax.jit
def _sc_call(logits):
  mesh = plsc.VectorSubcoreMesh(core_axis_name="c", subcore_axis_name="s")
  f = pl.kernel(
      _sc_body,
      out_type=(
          jax.ShapeDtypeStruct((BEAMS, L), jnp.float32),
          jax.ShapeDtypeStruct((BEAMS, K), jnp.float32),
          jax.ShapeDtypeStruct((BEAMS, K), jnp.int32),
      ),
      mesh=mesh,
      compiler_params=pltpu.CompilerParams(needs_layout_passes=False),
      scratch_types=[
          pltpu.VMEM((VOCAB,), jnp.float32),
          pltpu.VMEM((NBLK * L,), jnp.float32),
          pltpu.VMEM((CAP,), jnp.float32),
          pltpu.VMEM((CAP,), jnp.int32),
          pltpu.VMEM((NBLK + L,), jnp.int32),
          pltpu.VMEM((L,), jnp.float32),
          pltpu.VMEM((L,), jnp.int32),
          pltpu.VMEM((L,), jnp.float32),
          pltpu.SemaphoreType.DMA,
      ],
  )
  return f(logits, logits[:, VOCAB - 1024:].reshape(-1))


def _merge_body(hypo_ref, stats_ref, topv_ref, topi_ref,
                scores_ref, rows_ref, toks_ref, blank_ref):
  stats = stats_ref[...]
  m = stats[:, 0:1]
  s = stats[:, 1:2]
  xb = stats[:, 2:3]
  hypo = hypo_ref[...]
  lse = m + jnp.log(s)
  off = hypo - lse
  adj = topv_ref[...] + off
  blank_ref[...] = hypo + xb - lse

  rowi = lax.broadcasted_iota(jnp.int32, (BEAMS, K), 0)
  coli = lax.broadcasted_iota(jnp.int32, (BEAMS, K), 1)
  flat = rowi * K + coli
  topi = topi_ref[...]
  big = jnp.int32(BEAMS * K)
  lane16 = lax.broadcasted_iota(jnp.int32, (1, K), 1)
  sc = jnp.zeros((1, K), jnp.float32)
  ro = jnp.zeros((1, K), jnp.int32)
  tk = jnp.zeros((1, K), jnp.int32)
  for r in range(K):
    mv = jnp.max(adj)
    p = jnp.min(jnp.where(adj == mv, flat, big))
    row = p // K
    tok = jnp.sum(jnp.where(flat == p, topi, 0))
    sc = jnp.where(lane16 == r, mv, sc)
    ro = jnp.where(lane16 == r, row, ro)
    tk = jnp.where(lane16 == r, tok, tk)
    adj = jnp.where(flat == p, NEG, adj)
  scores_ref[...] = sc
  rows_ref[...] = ro
  toks_ref[...] = tk


@jax.jit
def _merge_call(hypo2, stats, topv, topi):
  return pl.pallas_call(
      _merge_body,
      out_shape=(
          jax.ShapeDtypeStruct((1, K), jnp.float32),
          jax.ShapeDtypeStruct((1, K), jnp.int32),
          jax.ShapeDtypeStruct((1, K), jnp.int32),
          jax.ShapeDtypeStruct((BEAMS, 1), jnp.float32),
      ),
  )(hypo2, stats, topv, topi)


def kernel(hypo_scores, logits, beam_width):
  del beam_width
  stats, topv, topi = _sc_call(logits)
  sc, ro, tk, blank = _merge_call(hypo_scores.reshape(BEAMS, 1),
                                  stats, topv, topi)
  return (sc.reshape(K), ro.reshape(K), tk.reshape(K), blank.reshape(BEAMS))
